# Initial kernel scaffold; baseline (speedup 1.0000x reference)
import jax
import jax.numpy as jnp
from jax.experimental import pallas as pl


def kernel(p, x, o, W1, gamma, beta, W2, b2):
    raise NotImplementedError("write your pallas kernel here")


# speedup vs baseline: 16.5247x; 16.5247x over previous
import functools

import jax
import jax.numpy as jnp
from jax.experimental import pallas as pl

_NS = 16
_EPS = 1e-5


def _qc_body(pp_ref, x_ref, w1p_ref, w1x_ref, q_ref, c_ref):
    c = jnp.dot(pp_ref[...], w1p_ref[...], preferred_element_type=jnp.float32)
    q_ref[...] = c + jnp.dot(x_ref[...], w1x_ref[...],
                             preferred_element_type=jnp.float32)
    c_ref[...] = c


def _knn_body(ns, s, ra, hid, pq_ref, pt_ref, qseg_ref, c_ref, h_ref,
              stats_ref):
    pq = pq_ref[...]
    pt = pt_ref[...]
    qseg = qseg_ref[...]
    c = c_ref[...]
    sqq = jnp.sum(pq * pq, axis=1, keepdims=True)
    sqk = jnp.sum(pt * pt, axis=0, keepdims=True)
    cross = jnp.dot(pq, pt, preferred_element_type=jnp.float32)
    d2 = sqq + sqk - 2.0 * cross
    iota = jax.lax.broadcasted_iota(jnp.int32, (ra, s), 1)
    acc1 = jnp.zeros((ra, hid), jnp.float32)
    acc2 = jnp.zeros((ra, hid), jnp.float32)
    for k in range(ns):
        m = jnp.min(d2, axis=1, keepdims=True)
        amin = jnp.min(jnp.where(d2 == m, iota, s), axis=1, keepdims=True)
        onehot = iota == amin
        hk = jnp.dot(onehot.astype(jnp.float32), qseg,
                     preferred_element_type=jnp.float32) - c
        h_ref[k] = hk
        acc1 = acc1 + hk
        acc2 = acc2 + hk * hk
        d2 = jnp.where(onehot, jnp.inf, d2)
    s1 = jnp.sum(acc1, axis=0, keepdims=True)
    s2 = jnp.sum(acc2, axis=0, keepdims=True)
    pad = jnp.zeros((6, hid), jnp.float32)
    stats_ref[0] = jnp.concatenate([s1, s2, pad], axis=0)


def _mlp_body(ns, rb, out_c, h_ref, sc_ref, sh_ref, w2_ref, b2_ref, out_ref):
    scale = sc_ref[...]
    shift = sh_ref[...]
    w2 = w2_ref[...]
    acc = jnp.full((rb, out_c), -jnp.inf, jnp.float32)
    for k in range(ns):
        z = jnp.maximum(h_ref[k] * scale + shift, 0.0)
        acc = jnp.maximum(acc, jnp.dot(z, w2,
                                       preferred_element_type=jnp.float32))
    out_ref[...] = acc + b2_ref[...]


def kernel(p, x, o, W1, gamma, beta, W2, b2):
    N, C = x.shape
    B = o.shape[0]
    S = N // B
    HID = W1.shape[1]
    OUT = W2.shape[1]
    NS = _NS
    RA = 256
    GA = N // RA
    RPB = S // RA
    RB = 512
    GB = N // RB

    pp = jnp.pad(p.astype(jnp.float32), ((0, 0), (0, 5)))
    pt = pp.T
    w1p = jnp.pad(W1[:3], ((0, 5), (0, 0)))
    w1x = W1[3:]

    q, c = pl.pallas_call(
        _qc_body,
        grid=(GA,),
        in_specs=[
            pl.BlockSpec((RA, 8), lambda i: (i, 0)),
            pl.BlockSpec((RA, C), lambda i: (i, 0)),
            pl.BlockSpec((8, HID), lambda i: (0, 0)),
            pl.BlockSpec((C, HID), lambda i: (0, 0)),
        ],
        out_specs=[
            pl.BlockSpec((RA, HID), lambda i: (i, 0)),
            pl.BlockSpec((RA, HID), lambda i: (i, 0)),
        ],
        out_shape=[
            jax.ShapeDtypeStruct((N, HID), jnp.float32),
            jax.ShapeDtypeStruct((N, HID), jnp.float32),
        ],
    )(pp, x, w1p, w1x)

    H, stats = pl.pallas_call(
        functools.partial(_knn_body, NS, S, RA, HID),
        grid=(GA,),
        in_specs=[
            pl.BlockSpec((RA, 8), lambda i: (i, 0)),
            pl.BlockSpec((8, S), lambda i: (0, i // RPB)),
            pl.BlockSpec((S, HID), lambda i: (i // RPB, 0)),
            pl.BlockSpec((RA, HID), lambda i: (i, 0)),
        ],
        out_specs=[
            pl.BlockSpec((NS, RA, HID), lambda i: (0, i, 0)),
            pl.BlockSpec((1, 8, HID), lambda i: (i, 0, 0)),
        ],
        out_shape=[
            jax.ShapeDtypeStruct((NS, N, HID), jnp.float32),
            jax.ShapeDtypeStruct((GA, 8, HID), jnp.float32),
        ],
    )(pp, pt, q, c)

    m = jnp.float32(N * NS)
    s1 = jnp.sum(stats[:, 0, :], axis=0)
    s2 = jnp.sum(stats[:, 1, :], axis=0)
    mean = s1 / m
    var = s2 / m - mean * mean
    scale = gamma / jnp.sqrt(var + _EPS)
    shift = beta - mean * scale

    out = pl.pallas_call(
        functools.partial(_mlp_body, NS, RB, OUT),
        grid=(GB,),
        in_specs=[
            pl.BlockSpec((NS, RB, HID), lambda i: (0, i, 0)),
            pl.BlockSpec((1, HID), lambda i: (0, 0)),
            pl.BlockSpec((1, HID), lambda i: (0, 0)),
            pl.BlockSpec((HID, OUT), lambda i: (0, 0)),
            pl.BlockSpec((1, OUT), lambda i: (0, 0)),
        ],
        out_specs=pl.BlockSpec((RB, OUT), lambda i: (i, 0)),
        out_shape=jax.ShapeDtypeStruct((N, OUT), jnp.float32),
    )(H, scale[None], shift[None], W2, b2[None])
    return out
